# baseline (speedup 1.0000x reference)
import functools
import math

import jax
import jax.numpy as jnp
from jax.experimental import pallas as pl

_NUM_EXPERTS = 64
_K = 8
_DIM = 768
_FREQ_DIM = 256
_B = 128
_HW = 16
_S = _HW * _HW
_NOISE_STD = 1.0 / _NUM_EXPERTS
_TAU = 1.0
_NEG = -1e30


def _pool_body(x_ref, o_ref):
    ones = jnp.full((_S, 1), 1.0, dtype=jnp.float32)
    s = jax.lax.dot_general(
        x_ref[...], ones,
        dimension_numbers=(((1,), (0,)), ((), ())),
        precision=jax.lax.Precision.HIGHEST,
        preferred_element_type=jnp.float32,
    )
    o_ref[...] = s * (1.0 / _S)


def _router_body(pooled_ref, freq_ref, wg_ref, wf_ref, comp_ref, noise_ref,
                 gates_ref, idx_ref, vals_ref, aux_ref):
    f32 = jnp.float32
    pooled = pooled_ref[...]
    logits = jax.lax.dot_general(
        pooled, wg_ref[...], (((1,), (1,)), ((), ())),
        precision=jax.lax.Precision.DEFAULT, preferred_element_type=f32)
    logits = logits + jax.lax.dot_general(
        freq_ref[...], wf_ref[...], (((1,), (1,)), ((), ())),
        precision=jax.lax.Precision.DEFAULT, preferred_element_type=f32)

    m = jnp.max(logits, axis=-1, keepdims=True)
    e = jnp.exp(logits - m)
    clean = e / jnp.sum(e, axis=-1, keepdims=True)
    importance = jnp.sum(clean, axis=0, keepdims=True) * comp_ref[...] * _TAU
    imp_mean = jnp.sum(importance, axis=1, keepdims=True) * (1.0 / _NUM_EXPERTS)
    imp_var = jnp.sum((importance - imp_mean) ** 2, axis=1, keepdims=True) * (
        1.0 / (_NUM_EXPERTS - 1))
    loss_imp = imp_var / (imp_mean + 1e-8) ** 2

    noisy = logits + noise_ref[...]
    m2 = jnp.max(noisy, axis=-1, keepdims=True)
    e2 = jnp.exp(noisy - m2)
    gprobs = e2 / jnp.sum(e2, axis=-1, keepdims=True)

    iota = jax.lax.broadcasted_iota(jnp.int32, (_B, _NUM_EXPERTS), 1)
    work = noisy
    gates = jnp.zeros((_B, _NUM_EXPERTS), f32)
    thr = None
    for k in range(_K):
        mk = jnp.max(work, axis=-1, keepdims=True)
        idxk = jnp.min(jnp.where(work == mk, iota, _NUM_EXPERTS),
                       axis=-1, keepdims=True)
        onehot = iota == idxk
        valk = jnp.sum(jnp.where(onehot, gprobs, 0.0), axis=-1, keepdims=True)
        gates = jnp.where(onehot, gprobs, gates)
        idx_ref[:, k:k + 1] = idxk
        vals_ref[:, k:k + 1] = valk
        work = jnp.where(onehot, _NEG, work)
        if k == _K - 1:
            thr = mk
    gates_ref[...] = gates

    inv_sqrt2 = 1.0 / math.sqrt(2.0)
    nr = (thr - logits) * (1.0 / _NOISE_STD)
    p = 1.0 - 0.5 * (1.0 + jax.lax.erf(nr * inv_sqrt2))
    p_mean = jnp.sum(p, axis=0, keepdims=True) * (1.0 / _B)
    pmm = jnp.sum(p_mean, axis=1, keepdims=True) * (1.0 / _NUM_EXPERTS)
    p_var = jnp.sum((p_mean - pmm) ** 2, axis=1, keepdims=True) * (
        1.0 / (_NUM_EXPERTS - 1))
    loss_load = p_var / (pmm + 1e-8) ** 2

    aux_ref[...] = 0.5 * loss_imp + 0.5 * loss_load


@functools.partial(jax.jit, static_argnames=("interpret",))
def _impl(x, freq_emb, W_gate, W_freq, complexity, interpret=False):
    noise = jax.random.normal(
        jax.random.key(1), (_B, _NUM_EXPERTS), dtype=jnp.float32) * _NOISE_STD

    x2 = x.reshape(_B * _DIM, _S)
    rows = _B * _DIM
    blk = 4096
    pooled = pl.pallas_call(
        _pool_body,
        grid=(rows // blk,),
        in_specs=[pl.BlockSpec((blk, _S), lambda i: (i, 0))],
        out_specs=pl.BlockSpec((blk, 1), lambda i: (i, 0)),
        out_shape=jax.ShapeDtypeStruct((rows, 1), jnp.float32),
        interpret=interpret,
    )(x2)
    pooled = pooled.reshape(_B, _DIM)

    comp2 = complexity.reshape(1, _NUM_EXPERTS)
    gates, idx, vals, aux = pl.pallas_call(
        _router_body,
        in_specs=[pl.BlockSpec(a.shape, lambda: (0,) * a.ndim)
                  for a in (pooled, freq_emb, W_gate, W_freq, comp2, noise)],
        out_specs=(
            pl.BlockSpec((_B, _NUM_EXPERTS), lambda: (0, 0)),
            pl.BlockSpec((_B, _K), lambda: (0, 0)),
            pl.BlockSpec((_B, _K), lambda: (0, 0)),
            pl.BlockSpec((1, 1), lambda: (0, 0)),
        ),
        out_shape=(
            jax.ShapeDtypeStruct((_B, _NUM_EXPERTS), jnp.float32),
            jax.ShapeDtypeStruct((_B, _K), jnp.int32),
            jax.ShapeDtypeStruct((_B, _K), jnp.float32),
            jax.ShapeDtypeStruct((1, 1), jnp.float32),
        ),
        interpret=interpret,
    )(pooled, freq_emb, W_gate, W_freq, comp2, noise)
    return gates, idx, vals, aux[0, 0]


def kernel(x, freq_emb, W_gate, W_freq, complexity):
    return _impl(x, freq_emb, W_gate, W_freq, complexity)


# speedup vs baseline: 8.8174x; 8.8174x over previous
import functools
import math

import jax
import jax.numpy as jnp
from jax.experimental import pallas as pl

_NUM_EXPERTS = 64
_K = 8
_DIM = 768
_FREQ_DIM = 256
_B = 128
_HW = 16
_S = _HW * _HW
_NOISE_STD = 1.0 / _NUM_EXPERTS
_TAU = 1.0
_NEG = -1e30


def _pool_body(x_ref, o_ref):
    o_ref[...] = jnp.sum(x_ref[...], axis=1) * (1.0 / _S)


def _router_body(pooled_ref, freq_ref, wg_ref, wf_ref, comp_ref, noise_ref,
                 gates_ref, idx_ref, vals_ref, aux_ref):
    f32 = jnp.float32
    pooled = pooled_ref[...]
    logits = jax.lax.dot_general(
        pooled, wg_ref[...], (((1,), (1,)), ((), ())),
        precision=jax.lax.Precision.DEFAULT, preferred_element_type=f32)
    logits = logits + jax.lax.dot_general(
        freq_ref[...], wf_ref[...], (((1,), (1,)), ((), ())),
        precision=jax.lax.Precision.DEFAULT, preferred_element_type=f32)

    m = jnp.max(logits, axis=-1, keepdims=True)
    e = jnp.exp(logits - m)
    clean = e / jnp.sum(e, axis=-1, keepdims=True)
    importance = jnp.sum(clean, axis=0, keepdims=True) * comp_ref[...] * _TAU
    imp_mean = jnp.sum(importance, axis=1, keepdims=True) * (1.0 / _NUM_EXPERTS)
    imp_var = jnp.sum((importance - imp_mean) ** 2, axis=1, keepdims=True) * (
        1.0 / (_NUM_EXPERTS - 1))
    loss_imp = imp_var / (imp_mean + 1e-8) ** 2

    noisy = logits + noise_ref[...]
    m2 = jnp.max(noisy, axis=-1, keepdims=True)
    e2 = jnp.exp(noisy - m2)
    gprobs = e2 / jnp.sum(e2, axis=-1, keepdims=True)

    iota = jax.lax.broadcasted_iota(jnp.int32, (_B, _NUM_EXPERTS), 1)
    work = noisy
    gates = jnp.zeros((_B, _NUM_EXPERTS), f32)
    thr = None
    for k in range(_K):
        mk = jnp.max(work, axis=-1, keepdims=True)
        idxk = jnp.min(jnp.where(work == mk, iota, _NUM_EXPERTS),
                       axis=-1, keepdims=True)
        onehot = iota == idxk
        valk = jnp.sum(jnp.where(onehot, gprobs, 0.0), axis=-1, keepdims=True)
        gates = jnp.where(onehot, gprobs, gates)
        idx_ref[:, k:k + 1] = idxk
        vals_ref[:, k:k + 1] = valk
        work = jnp.where(onehot, _NEG, work)
        if k == _K - 1:
            thr = mk
    gates_ref[...] = gates

    inv_sqrt2 = 1.0 / math.sqrt(2.0)
    nr = (thr - logits) * (1.0 / _NOISE_STD)
    p = 1.0 - 0.5 * (1.0 + jax.lax.erf(nr * inv_sqrt2))
    p_mean = jnp.sum(p, axis=0, keepdims=True) * (1.0 / _B)
    pmm = jnp.sum(p_mean, axis=1, keepdims=True) * (1.0 / _NUM_EXPERTS)
    p_var = jnp.sum((p_mean - pmm) ** 2, axis=1, keepdims=True) * (
        1.0 / (_NUM_EXPERTS - 1))
    loss_load = p_var / (pmm + 1e-8) ** 2

    aux_ref[...] = 0.5 * loss_imp + 0.5 * loss_load


@functools.partial(jax.jit, static_argnames=("interpret",))
def _impl(x, freq_emb, W_gate, W_freq, complexity, interpret=False):
    noise = jax.random.normal(
        jax.random.key(1), (_B, _NUM_EXPERTS), dtype=jnp.float32) * _NOISE_STD

    xt = jnp.transpose(x, (0, 2, 3, 1)).reshape(_B, _S, _DIM)
    pb = 8
    pooled = pl.pallas_call(
        _pool_body,
        grid=(_B // pb,),
        in_specs=[pl.BlockSpec((pb, _S, _DIM), lambda i: (i, 0, 0))],
        out_specs=pl.BlockSpec((pb, _DIM), lambda i: (i, 0)),
        out_shape=jax.ShapeDtypeStruct((_B, _DIM), jnp.float32),
        interpret=interpret,
    )(xt)

    comp2 = complexity.reshape(1, _NUM_EXPERTS)
    gates, idx, vals, aux = pl.pallas_call(
        _router_body,
        in_specs=[pl.BlockSpec(a.shape, lambda: (0,) * a.ndim)
                  for a in (pooled, freq_emb, W_gate, W_freq, comp2, noise)],
        out_specs=(
            pl.BlockSpec((_B, _NUM_EXPERTS), lambda: (0, 0)),
            pl.BlockSpec((_B, _K), lambda: (0, 0)),
            pl.BlockSpec((_B, _K), lambda: (0, 0)),
            pl.BlockSpec((1, 1), lambda: (0, 0)),
        ),
        out_shape=(
            jax.ShapeDtypeStruct((_B, _NUM_EXPERTS), jnp.float32),
            jax.ShapeDtypeStruct((_B, _K), jnp.int32),
            jax.ShapeDtypeStruct((_B, _K), jnp.float32),
            jax.ShapeDtypeStruct((1, 1), jnp.float32),
        ),
        interpret=interpret,
    )(pooled, freq_emb, W_gate, W_freq, comp2, noise)
    return gates, idx, vals, aux[0, 0]


def kernel(x, freq_emb, W_gate, W_freq, complexity):
    return _impl(x, freq_emb, W_gate, W_freq, complexity)
